# baseline (speedup 1.0000x reference)
import functools

import jax
import jax.numpy as jnp
from jax import lax
from jax.experimental import pallas as pl
from jax.experimental.pallas import tpu as pltpu
from jax.experimental.pallas import tpu_sc as plsc

B = 32
H = 1024
T = 2048
DEPTH = 28
K = 512

TBLK = 256
SBLK = 128
HBLK = 1024

_NOISE_CACHE = []


def _noise_const():
    if not _NOISE_CACHE:
        _NOISE_CACHE.append(
            jax.random.normal(jax.random.key(1234), (B, T), jnp.float32)
        )
    return _NOISE_CACHE[0]


def _fused_body(t_ref, w1_ref, b1_ref, w2_ref, b2_ref, wg_ref, wgl_ref,
                bg_ref, wn_ref, nz_ref, step_ref, nz_out_ref, te_out_ref,
                te_scratch):
    @pl.when(pl.program_id(0) == 0)
    def _():
        h = t_ref[...] * w1_ref[...] + b1_ref[...]
        h = h * jax.nn.sigmoid(h)
        te = (
            jnp.dot(
                h.astype(jnp.bfloat16),
                w2_ref[...].astype(jnp.bfloat16),
                preferred_element_type=jnp.float32,
            )
            + b2_ref[...]
        )
        te_scratch[...] = te
        te_out_ref[...] = te

    te = te_scratch[...].astype(jnp.bfloat16)
    step_b = step_ref[...].astype(jnp.bfloat16).astype(jnp.float32)
    wgl_b = wgl_ref[...].astype(jnp.bfloat16).astype(jnp.float32)
    clean = (
        jnp.dot(te, wg_ref[...].astype(jnp.bfloat16), preferred_element_type=jnp.float32)
        + step_b * wgl_b
        + bg_ref[...]
    )
    raw = jnp.dot(te, wn_ref[...].astype(jnp.bfloat16), preferred_element_type=jnp.float32)
    std = jnp.maximum(raw, 0.0) + jnp.log1p(jnp.exp(-jnp.abs(raw))) + 0.01
    nz_out_ref[...] = clean + nz_ref[...] * std


def _noisy_logits(t, W1, b1r, W2, b2r, Wg_main, wg_last, bgr, w_noise, noise,
                  step_f):
    grid = (T // TBLK,)
    return pl.pallas_call(
        _fused_body,
        grid=grid,
        in_specs=[
            pl.BlockSpec((B, 1), lambda i: (0, 0)),
            pl.BlockSpec((1, H), lambda i: (0, 0)),
            pl.BlockSpec((1, H), lambda i: (0, 0)),
            pl.BlockSpec((H, H), lambda i: (0, 0)),
            pl.BlockSpec((1, H), lambda i: (0, 0)),
            pl.BlockSpec((H, TBLK), lambda i: (0, i)),
            pl.BlockSpec((1, TBLK), lambda i: (0, i)),
            pl.BlockSpec((1, TBLK), lambda i: (0, i)),
            pl.BlockSpec((H, TBLK), lambda i: (0, i)),
            pl.BlockSpec((B, TBLK), lambda i: (0, i)),
            pl.BlockSpec((1, 1), lambda i: (0, 0)),
        ],
        out_specs=[
            pl.BlockSpec((B, TBLK), lambda i: (0, i)),
            pl.BlockSpec((B, H), lambda i: (0, 0)),
        ],
        out_shape=[
            jax.ShapeDtypeStruct((B, T), jnp.float32),
            jax.ShapeDtypeStruct((B, H), jnp.float32),
        ],
        scratch_shapes=[pltpu.VMEM((B, H), jnp.float32)],
    )(t, W1, b1r, W2, b2r, Wg_main, wg_last, bgr, w_noise, noise, step_f)


def _gates_body(x_ref, out_ref):
    x = x_ref[...]
    s = lax.bitcast_convert_type(x, jnp.int32)
    m = lax.shift_right_arithmetic(s, 31)
    keys = lax.bitcast_convert_type(
        s ^ (m & jnp.int32(0x7FFFFFFF)) ^ jnp.int32(-0x80000000), jnp.uint32
    )

    def bit_step(i, t):
        cand = t | (jnp.uint32(1) << (jnp.uint32(31) - i.astype(jnp.uint32)))
        cnt = jnp.sum((keys >= cand).astype(jnp.int32), axis=1, keepdims=True)
        return jnp.where(cnt >= K, cand, t)

    thr = lax.fori_loop(0, 32, bit_step, jnp.zeros((B, 1), jnp.uint32))

    gt = keys > thr
    eq = keys == thr
    n_gt = jnp.sum(gt.astype(jnp.int32), axis=1, keepdims=True)
    tie_rank = eq.astype(jnp.int32)
    d = 1
    while d < T:
        shifted = jnp.concatenate(
            [jnp.zeros((B, d), jnp.int32), tie_rank[:, : T - d]], axis=1
        )
        tie_rank = tie_rank + shifted
        d *= 2
    sel = gt | (eq & (tie_rank <= (K - n_gt)))

    mx = jnp.max(x, axis=1, keepdims=True)
    e = jnp.where(sel, jnp.exp(x - mx), 0.0)
    out_ref[...] = e / jnp.sum(e, axis=1, keepdims=True)


def _gates(noisy):
    return pl.pallas_call(
        _gates_body,
        out_shape=jax.ShapeDtypeStruct((B, T), jnp.float32),
    )(noisy)


_NV = T // 16


def _lane_iota():
    return lax.iota(jnp.int32, 16)


def _bfly_sum_i32(v):
    for d in (1, 2, 4, 8):
        v = v + jnp.take(v, _lane_iota() ^ d, axis=0)
    return v


def _bfly_sum_f32(v):
    for d in (1, 2, 4, 8):
        v = v + jnp.take(v, _lane_iota() ^ d, axis=0)
    return v


def _bfly_max_f32(v):
    for d in (1, 2, 4, 8):
        v = jnp.maximum(v, jnp.take(v, _lane_iota() ^ d, axis=0))
    return v


def _prefix_sum_i32(p):
    io = _lane_iota()
    for d in (1, 2, 4, 8):
        shifted = jnp.take(p, jnp.maximum(io - d, 0), axis=0)
        p = p + jnp.where(io >= d, shifted, 0)
    return p


def _gates_sc_body(x_hbm, out_hbm, row_v, keys_v, g_v):
    wid = lax.axis_index("s") * 2 + lax.axis_index("c")
    pltpu.sync_copy(x_hbm.at[wid], row_v)

    def mk_keys(j, mx):
        v = row_v[pl.ds(j * 16, 16)]
        s = lax.bitcast_convert_type(v, jnp.int32)
        m = lax.shift_right_arithmetic(s, 31)
        keys_v[pl.ds(j * 16, 16)] = lax.bitcast_convert_type(
            s ^ (m & jnp.int32(0x7FFFFFFF)) ^ jnp.int32(-0x80000000), jnp.uint32
        )
        return jnp.maximum(mx, v)

    mxv = lax.fori_loop(0, _NV, mk_keys, jnp.full((16,), -3.0e38, jnp.float32),
                        unroll=8)
    mx = _bfly_max_f32(mxv)

    def count_ge(cand):
        def cnt_step(j, acc):
            kv = keys_v[pl.ds(j * 16, 16)]
            return acc + jnp.where(kv >= cand, 1, 0).astype(jnp.int32)

        acc = lax.fori_loop(0, _NV, cnt_step, jnp.zeros((16,), jnp.int32),
                            unroll=8)
        return _bfly_sum_i32(acc)[0]

    def bit_step(i, t):
        cand = t | (jnp.uint32(1) << (jnp.uint32(31) - i.astype(jnp.uint32)))
        return jnp.where(count_ge(cand) >= K, cand, t)

    thr = lax.fori_loop(0, 32, bit_step, jnp.uint32(0))

    n_gt = count_ge(thr + jnp.uint32(1))
    need = K - n_gt

    def sel_step(j, carry):
        tie_seen, s_acc = carry
        kv = keys_v[pl.ds(j * 16, 16)]
        v = row_v[pl.ds(j * 16, 16)]
        eq = kv == thr
        eq_i = jnp.where(eq, 1, 0).astype(jnp.int32)
        pre = _prefix_sum_i32(eq_i)
        rank = tie_seen + pre
        sel = (kv > thr) | (eq & (rank <= need))
        e = jnp.where(sel, jnp.exp(v - mx), 0.0)
        g_v[pl.ds(j * 16, 16)] = e
        return tie_seen + pre[15], s_acc + e

    _, s_acc = lax.fori_loop(
        0, _NV, sel_step,
        (jnp.int32(0), jnp.zeros((16,), jnp.float32)), unroll=8)
    inv = 1.0 / _bfly_sum_f32(s_acc)

    def scale_step(j, _):
        g_v[pl.ds(j * 16, 16)] = g_v[pl.ds(j * 16, 16)] * inv
        return 0

    lax.fori_loop(0, _NV, scale_step, 0, unroll=8)
    pltpu.sync_copy(g_v, out_hbm.at[wid])


def _gates_sc(noisy):
    f = functools.partial(
        pl.kernel,
        mesh=plsc.VectorSubcoreMesh(core_axis_name="c", subcore_axis_name="s"),
        out_type=jax.ShapeDtypeStruct((B, T), jnp.float32),
        scratch_types=[
            pltpu.VMEM((T,), jnp.float32),
            pltpu.VMEM((T,), jnp.uint32),
            pltpu.VMEM((T,), jnp.float32),
        ],
    )(_gates_sc_body)
    return f(noisy)


def _scale_body(step_ref, p_ref, g_ref, out_ref):
    del step_ref
    out_ref[...] = g_ref[...][:, :, None] * p_ref[...]


def _scale(timestep, prompt_embeddings, gates):
    grid = (T // SBLK, H // HBLK)
    return pl.pallas_call(
        _scale_body,
        grid_spec=pltpu.PrefetchScalarGridSpec(
            num_scalar_prefetch=1,
            grid=grid,
            in_specs=[
                pl.BlockSpec((1, SBLK, HBLK), lambda i, j, step: (step[0], i, j)),
                pl.BlockSpec((B, SBLK), lambda i, j, step: (0, i)),
            ],
            out_specs=pl.BlockSpec((B, SBLK, HBLK), lambda i, j, step: (0, i, j)),
        ),
        out_shape=jax.ShapeDtypeStruct((B, T, H), jnp.float32),
    )(timestep, prompt_embeddings, gates)


def kernel(timestep, prompt_embeddings, W1, b1, W2, b2, Wg, bg, w_noise):
    t = timestep.astype(jnp.float32).reshape(B, 1)
    step_f = timestep[0].astype(jnp.float32).reshape(1, 1)

    noisy, t_embed = _noisy_logits(
        t, W1, b1.reshape(1, H), W2, b2.reshape(1, H),
        Wg, Wg[H:], bg.reshape(1, T), w_noise, _noise_const(), step_f
    )
    gates = _gates_sc(noisy)
    prompts = _scale(timestep, prompt_embeddings, gates)
    return prompts, t_embed


# speedup vs baseline: 1.0031x; 1.0031x over previous
import functools

import jax
import jax.numpy as jnp
from jax import lax
from jax.experimental import pallas as pl
from jax.experimental.pallas import tpu as pltpu
from jax.experimental.pallas import tpu_sc as plsc

B = 32
H = 1024
T = 2048
DEPTH = 28
K = 512

TBLK = 256
SBLK = 128
HBLK = 1024


def _fused_body(t_ref, w1_ref, b1_ref, w2_ref, b2_ref, wg_ref, wgl_ref,
                bg_ref, wn_ref, nz_ref, step_ref, nz_out_ref, te_out_ref,
                te_scratch):
    @pl.when(pl.program_id(0) == 0)
    def _():
        h = t_ref[...] * w1_ref[...] + b1_ref[...]
        h = h * jax.nn.sigmoid(h)
        te = (
            jnp.dot(
                h.astype(jnp.bfloat16),
                w2_ref[...].astype(jnp.bfloat16),
                preferred_element_type=jnp.float32,
            )
            + b2_ref[...]
        )
        te_scratch[...] = te
        te_out_ref[...] = te

    te = te_scratch[...].astype(jnp.bfloat16)
    step_b = step_ref[...].astype(jnp.bfloat16).astype(jnp.float32)
    wgl_b = wgl_ref[...].astype(jnp.bfloat16).astype(jnp.float32)
    clean = (
        jnp.dot(te, wg_ref[...].astype(jnp.bfloat16), preferred_element_type=jnp.float32)
        + step_b * wgl_b
        + bg_ref[...]
    )
    raw = jnp.dot(te, wn_ref[...].astype(jnp.bfloat16), preferred_element_type=jnp.float32)
    std = jnp.maximum(raw, 0.0) + jnp.log1p(jnp.exp(-jnp.abs(raw))) + 0.01
    nz_out_ref[...] = clean + nz_ref[...] * std


def _noisy_logits(t, W1, b1r, W2, b2r, Wg_main, wg_last, bgr, w_noise, noise,
                  step_f):
    grid = (T // TBLK,)
    return pl.pallas_call(
        _fused_body,
        grid=grid,
        in_specs=[
            pl.BlockSpec((B, 1), lambda i: (0, 0)),
            pl.BlockSpec((1, H), lambda i: (0, 0)),
            pl.BlockSpec((1, H), lambda i: (0, 0)),
            pl.BlockSpec((H, H), lambda i: (0, 0)),
            pl.BlockSpec((1, H), lambda i: (0, 0)),
            pl.BlockSpec((H, TBLK), lambda i: (0, i)),
            pl.BlockSpec((1, TBLK), lambda i: (0, i)),
            pl.BlockSpec((1, TBLK), lambda i: (0, i)),
            pl.BlockSpec((H, TBLK), lambda i: (0, i)),
            pl.BlockSpec((B, TBLK), lambda i: (0, i)),
            pl.BlockSpec((1, 1), lambda i: (0, 0)),
        ],
        out_specs=[
            pl.BlockSpec((B, TBLK), lambda i: (0, i)),
            pl.BlockSpec((B, H), lambda i: (0, 0)),
        ],
        out_shape=[
            jax.ShapeDtypeStruct((B, T), jnp.float32),
            jax.ShapeDtypeStruct((B, H), jnp.float32),
        ],
        scratch_shapes=[pltpu.VMEM((B, H), jnp.float32)],
    )(t, W1, b1r, W2, b2r, Wg_main, wg_last, bgr, w_noise, noise, step_f)


def _gates_body(x_ref, out_ref):
    x = x_ref[...]
    s = lax.bitcast_convert_type(x, jnp.int32)
    m = lax.shift_right_arithmetic(s, 31)
    keys = lax.bitcast_convert_type(
        s ^ (m & jnp.int32(0x7FFFFFFF)) ^ jnp.int32(-0x80000000), jnp.uint32
    )

    def bit_step(i, t):
        cand = t | (jnp.uint32(1) << (jnp.uint32(31) - i.astype(jnp.uint32)))
        cnt = jnp.sum((keys >= cand).astype(jnp.int32), axis=1, keepdims=True)
        return jnp.where(cnt >= K, cand, t)

    thr = lax.fori_loop(0, 32, bit_step, jnp.zeros((B, 1), jnp.uint32))

    gt = keys > thr
    eq = keys == thr
    n_gt = jnp.sum(gt.astype(jnp.int32), axis=1, keepdims=True)
    tie_rank = eq.astype(jnp.int32)
    d = 1
    while d < T:
        shifted = jnp.concatenate(
            [jnp.zeros((B, d), jnp.int32), tie_rank[:, : T - d]], axis=1
        )
        tie_rank = tie_rank + shifted
        d *= 2
    sel = gt | (eq & (tie_rank <= (K - n_gt)))

    mx = jnp.max(x, axis=1, keepdims=True)
    e = jnp.where(sel, jnp.exp(x - mx), 0.0)
    out_ref[...] = e / jnp.sum(e, axis=1, keepdims=True)


def _gates(noisy):
    return pl.pallas_call(
        _gates_body,
        out_shape=jax.ShapeDtypeStruct((B, T), jnp.float32),
    )(noisy)


_NV = T // 16


def _lane_iota():
    return lax.iota(jnp.int32, 16)


def _bfly_sum_i32(v):
    for d in (1, 2, 4, 8):
        v = v + jnp.take(v, _lane_iota() ^ d, axis=0)
    return v


def _bfly_sum_f32(v):
    for d in (1, 2, 4, 8):
        v = v + jnp.take(v, _lane_iota() ^ d, axis=0)
    return v


def _bfly_max_f32(v):
    for d in (1, 2, 4, 8):
        v = jnp.maximum(v, jnp.take(v, _lane_iota() ^ d, axis=0))
    return v


def _prefix_sum_i32(p):
    io = _lane_iota()
    for d in (1, 2, 4, 8):
        shifted = jnp.take(p, jnp.maximum(io - d, 0), axis=0)
        p = p + jnp.where(io >= d, shifted, 0)
    return p


def _gates_sc_body(x_hbm, out_hbm, sum_hbm, row_v, keys_v, g_v, sum_v):
    wid = lax.axis_index("s") * 2 + lax.axis_index("c")
    pltpu.sync_copy(x_hbm.at[wid], row_v)

    def mk_keys(j, mx):
        v = row_v[pl.ds(j * 16, 16)]
        s = lax.bitcast_convert_type(v, jnp.int32)
        m = lax.shift_right_arithmetic(s, 31)
        keys_v[pl.ds(j * 16, 16)] = lax.bitcast_convert_type(
            s ^ (m & jnp.int32(0x7FFFFFFF)) ^ jnp.int32(-0x80000000), jnp.uint32
        )
        return jnp.maximum(mx, v)

    mxv = lax.fori_loop(0, _NV, mk_keys, jnp.full((16,), -3.0e38, jnp.float32),
                        unroll=8)
    mx = _bfly_max_f32(mxv)

    def count_ge(cand):
        def cnt_step(j, acc):
            kv = keys_v[pl.ds(j * 16, 16)]
            return acc + jnp.where(kv >= cand, 1, 0).astype(jnp.int32)

        acc = lax.fori_loop(0, _NV, cnt_step, jnp.zeros((16,), jnp.int32),
                            unroll=16)
        return _bfly_sum_i32(acc)[0]

    def bit_step(i, t):
        cand = t | (jnp.uint32(1) << (jnp.uint32(31) - i.astype(jnp.uint32)))
        return jnp.where(count_ge(cand) >= K, cand, t)

    thr = lax.fori_loop(0, 32, bit_step, jnp.uint32(0))

    n_gt = count_ge(thr + jnp.uint32(1))
    need = K - n_gt

    def sel_step(j, carry):
        tie_seen, s_acc = carry
        kv = keys_v[pl.ds(j * 16, 16)]
        v = row_v[pl.ds(j * 16, 16)]
        eq = kv == thr
        eq_i = jnp.where(eq, 1, 0).astype(jnp.int32)
        pre = _prefix_sum_i32(eq_i)
        rank = tie_seen + pre
        sel = (kv > thr) | (eq & (rank <= need))
        e = jnp.where(sel, jnp.exp(v - mx), 0.0)
        g_v[pl.ds(j * 16, 16)] = e
        return tie_seen + pre[15], s_acc + e

    _, s_acc = lax.fori_loop(
        0, _NV, sel_step,
        (jnp.int32(0), jnp.zeros((16,), jnp.float32)), unroll=8)
    sum_v[...] = _bfly_sum_f32(s_acc)
    pltpu.sync_copy(g_v, out_hbm.at[wid])
    pltpu.sync_copy(sum_v, sum_hbm.at[wid])


def _gates_sc(noisy):
    f = functools.partial(
        pl.kernel,
        mesh=plsc.VectorSubcoreMesh(core_axis_name="c", subcore_axis_name="s"),
        out_type=[
            jax.ShapeDtypeStruct((B, T), jnp.float32),
            jax.ShapeDtypeStruct((B, 16), jnp.float32),
        ],
        scratch_types=[
            pltpu.VMEM((T,), jnp.float32),
            pltpu.VMEM((T,), jnp.uint32),
            pltpu.VMEM((T,), jnp.float32),
            pltpu.VMEM((16,), jnp.float32),
        ],
    )(_gates_sc_body)
    return f(noisy)


def _scale_body(step_ref, p_ref, g_ref, s_ref, out_ref):
    del step_ref
    g = g_ref[...] / s_ref[:, :1]
    out_ref[...] = g[:, :, None] * p_ref[...]


def _scale(timestep, prompt_embeddings, gates_e, gate_sums):
    grid = (T // SBLK, H // HBLK)
    return pl.pallas_call(
        _scale_body,
        grid_spec=pltpu.PrefetchScalarGridSpec(
            num_scalar_prefetch=1,
            grid=grid,
            in_specs=[
                pl.BlockSpec((1, SBLK, HBLK), lambda i, j, step: (step[0], i, j)),
                pl.BlockSpec((B, SBLK), lambda i, j, step: (0, i)),
                pl.BlockSpec((B, 16), lambda i, j, step: (0, 0)),
            ],
            out_specs=pl.BlockSpec((B, SBLK, HBLK), lambda i, j, step: (0, i, j)),
        ),
        out_shape=jax.ShapeDtypeStruct((B, T, H), jnp.float32),
    )(timestep, prompt_embeddings, gates_e, gate_sums)


def kernel(timestep, prompt_embeddings, W1, b1, W2, b2, Wg, bg, w_noise):
    t = timestep.astype(jnp.float32).reshape(B, 1)
    step_f = timestep[0].astype(jnp.float32).reshape(1, 1)
    noise = jax.random.normal(jax.random.key(1234), (B, T), jnp.float32)

    noisy, t_embed = _noisy_logits(
        t, W1, b1.reshape(1, H), W2, b2.reshape(1, H),
        Wg, Wg[H:], bg.reshape(1, T), w_noise, noise, step_f
    )
    gates_e, gate_sums = _gates_sc(noisy)
    prompts = _scale(timestep, prompt_embeddings, gates_e, gate_sums)
    return prompts, t_embed
